# baseline (speedup 1.0000x reference)
import jax
import jax.numpy as jnp
from jax.experimental import pallas as pl
from jax.experimental.pallas import tpu as pltpu
from jax.experimental.pallas import tpu_sc as plsc

BS, LENGTH, D_Y, D_X, D_MODEL = 8, 512, 8, 7, 512
T2V_IN = D_X + 1
T2V_K = D_MODEL // T2V_IN
N_TOK = D_Y * LENGTH
REP = 64


def _tc_body(xc_ref, y_ref, e_ref, w_ref, b_ref, w0_ref, w1_ref, c_ref,
             out_ref):
    xc = xc_ref[0]
    xce = jax.lax.dot(xc, e_ref[...], precision=jax.lax.Precision.HIGHEST)
    a = xce * w_ref[...] + b_ref[...]
    ids = jax.lax.broadcasted_iota(jnp.int32, (LENGTH, D_MODEL), 1)
    s = jnp.where((ids & (T2V_K - 1)) == 0, a, jnp.sin(a))
    t = jax.lax.dot(s, w1_ref[...],
                    precision=jax.lax.Precision.HIGHEST) + c_ref[...]
    yb = y_ref[0]
    for v in range(D_Y):
        yv = yb[:, v:v + 1]
        out_ref[0, v * LENGTH:(v + 1) * LENGTH, :] = t + yv * w0_ref[...]


def _sc_body(tab_ref, vemb_ref, vidx_ref, rep_ref, idx_ref, gidx_ref, sem):
    core = jax.lax.axis_index("c")
    sub = jax.lax.axis_index("s")
    tec = core * 16 + sub
    for j in range(2):
        blk = tec * 2 + j
        b = blk // D_Y
        v = blk % D_Y
        vvec = jnp.broadcast_to(v, (16,)).astype(jnp.int32)
        for i in range(REP // 16):
            gidx_ref[pl.ds(i * 16, 16)] = vvec
        pltpu.async_copy(tab_ref.at[gidx_ref], rep_ref, sem).wait()

        @pl.loop(0, LENGTH, step=16)
        def _(i):
            idx_ref[pl.ds(i, 16)] = vvec

        copies = [
            pltpu.async_copy(
                rep_ref,
                vemb_ref.at[b, pl.ds(v * LENGTH + k * REP, REP), :], sem)
            for k in range(LENGTH // REP)
        ]
        copies.append(
            pltpu.async_copy(idx_ref,
                             vidx_ref.at[b, pl.ds(v * LENGTH, LENGTH)], sem))
        for cp in copies:
            cp.wait()


def _sc_var_outputs(var_emb_table):
    mesh = plsc.VectorSubcoreMesh(core_axis_name="c", subcore_axis_name="s")
    fn = pl.kernel(
        _sc_body,
        out_type=[
            jax.ShapeDtypeStruct((BS, N_TOK, D_MODEL), jnp.float32),
            jax.ShapeDtypeStruct((BS, N_TOK), jnp.int32),
        ],
        mesh=mesh,
        scratch_types=[
            pltpu.VMEM((REP, D_MODEL), jnp.float32),
            pltpu.VMEM((LENGTH,), jnp.int32),
            pltpu.VMEM((REP,), jnp.int32),
            pltpu.SemaphoreType.DMA,
        ],
    )
    return fn(var_emb_table)


def kernel(y, x, t2v_weight, t2v_bias, y_emb_W, y_emb_b, var_emb_table,
           given_emb_table):
    local_pos = jnp.broadcast_to(
        (jnp.arange(LENGTH, dtype=jnp.float32) / LENGTH)[None, :, None],
        (BS, LENGTH, 1))
    xc = jnp.concatenate([x, local_pos], axis=-1)
    e = jnp.repeat(jnp.eye(T2V_IN, dtype=jnp.float32), T2V_K, axis=1)
    wrow = t2v_weight.reshape(1, D_MODEL)
    brow = t2v_bias.reshape(1, D_MODEL)
    w0 = y_emb_W[0:1]
    w1 = y_emb_W[1:]
    c = (y_emb_b + given_emb_table[1])[None]

    val_time = pl.pallas_call(
        _tc_body,
        grid=(BS,),
        in_specs=[
            pl.BlockSpec((1, LENGTH, T2V_IN), lambda b: (b, 0, 0)),
            pl.BlockSpec((1, LENGTH, D_Y), lambda b: (b, 0, 0)),
            pl.BlockSpec((T2V_IN, D_MODEL), lambda b: (0, 0)),
            pl.BlockSpec((1, D_MODEL), lambda b: (0, 0)),
            pl.BlockSpec((1, D_MODEL), lambda b: (0, 0)),
            pl.BlockSpec((1, D_MODEL), lambda b: (0, 0)),
            pl.BlockSpec((D_MODEL, D_MODEL), lambda b: (0, 0)),
            pl.BlockSpec((1, D_MODEL), lambda b: (0, 0)),
        ],
        out_specs=pl.BlockSpec((1, N_TOK, D_MODEL), lambda b: (b, 0, 0)),
        out_shape=jax.ShapeDtypeStruct((BS, N_TOK, D_MODEL), jnp.float32),
    )(xc, y, e, wrow, brow, w0, w1, c)
    var_emb, var_idx = _sc_var_outputs(var_emb_table)
    return val_time, var_emb, var_idx


# speedup vs baseline: 1.7415x; 1.7415x over previous
import jax
import jax.numpy as jnp
from jax.experimental import pallas as pl
from jax.experimental.pallas import tpu as pltpu
from jax.experimental.pallas import tpu_sc as plsc

BS, LENGTH, D_Y, D_X, D_MODEL = 8, 512, 8, 7, 512
T2V_IN = D_X + 1
T2V_K = D_MODEL // T2V_IN
N_TOK = D_Y * LENGTH
REP = 128


def _tc_body(xc_ref, y_ref, e_ref, w_ref, b_ref, w0_ref, w1_ref, c_ref,
             out_ref):
    xc = xc_ref[0]
    xce = jax.lax.dot(xc, e_ref[...], precision=jax.lax.Precision.HIGHEST)
    a = xce * w_ref[...] + b_ref[...]
    ids = jax.lax.broadcasted_iota(jnp.int32, (LENGTH, D_MODEL), 1)
    s = jnp.where((ids & (T2V_K - 1)) == 0, a, jnp.sin(a))
    t = jax.lax.dot(s, w1_ref[...],
                    precision=jax.lax.Precision.HIGHEST) + c_ref[...]
    yb = y_ref[0]
    for v in range(D_Y):
        yv = yb[:, v:v + 1]
        out_ref[0, v * LENGTH:(v + 1) * LENGTH, :] = t + yv * w0_ref[...]


def _sc_body(tab_ref, vemb_ref, vidx_ref, rep_ref, idx_ref, sem):
    core = jax.lax.axis_index("c")
    sub = jax.lax.axis_index("s")
    tec = core * 16 + sub
    v = jax.lax.rem(tec, D_Y)
    b0 = jax.lax.div(tec, D_Y)
    pltpu.async_copy(tab_ref.at[v], rep_ref.at[0], sem).wait()
    vvec = jnp.broadcast_to(v, (16,)).astype(jnp.int32)

    @pl.loop(0, D_MODEL, step=16)
    def _(i):
        val = rep_ref[0, pl.ds(i, 16)]
        for r in range(1, REP):
            rep_ref[r, pl.ds(i, 16)] = val

    @pl.loop(0, LENGTH, step=16)
    def _(i):
        idx_ref[pl.ds(i, 16)] = vvec

    copies = []
    for j in range(2):
        b = b0 + j * 4
        for k in range(LENGTH // REP):
            copies.append(pltpu.async_copy(
                rep_ref,
                vemb_ref.at[b, pl.ds(v * LENGTH + k * REP, REP), :], sem))
        copies.append(pltpu.async_copy(
            idx_ref, vidx_ref.at[b, pl.ds(v * LENGTH, LENGTH)], sem))
    for cp in copies:
        cp.wait()


def _sc_var_outputs(var_emb_table):
    mesh = plsc.VectorSubcoreMesh(core_axis_name="c", subcore_axis_name="s")
    fn = pl.kernel(
        _sc_body,
        out_type=[
            jax.ShapeDtypeStruct((BS, N_TOK, D_MODEL), jnp.float32),
            jax.ShapeDtypeStruct((BS, N_TOK), jnp.int32),
        ],
        mesh=mesh,
        scratch_types=[
            pltpu.VMEM((REP, D_MODEL), jnp.float32),
            pltpu.VMEM((LENGTH,), jnp.int32),
            pltpu.SemaphoreType.DMA,
        ],
    )
    return fn(var_emb_table)


def kernel(y, x, t2v_weight, t2v_bias, y_emb_W, y_emb_b, var_emb_table,
           given_emb_table):
    local_pos = jnp.broadcast_to(
        (jnp.arange(LENGTH, dtype=jnp.float32) / LENGTH)[None, :, None],
        (BS, LENGTH, 1))
    xc = jnp.concatenate([x, local_pos], axis=-1)
    e = jnp.repeat(jnp.eye(T2V_IN, dtype=jnp.float32), T2V_K, axis=1)
    wrow = t2v_weight.reshape(1, D_MODEL)
    brow = t2v_bias.reshape(1, D_MODEL)
    w0 = y_emb_W[0:1]
    w1 = y_emb_W[1:]
    c = (y_emb_b + given_emb_table[1])[None]

    val_time = pl.pallas_call(
        _tc_body,
        grid=(BS,),
        in_specs=[
            pl.BlockSpec((1, LENGTH, T2V_IN), lambda b: (b, 0, 0)),
            pl.BlockSpec((1, LENGTH, D_Y), lambda b: (b, 0, 0)),
            pl.BlockSpec((T2V_IN, D_MODEL), lambda b: (0, 0)),
            pl.BlockSpec((1, D_MODEL), lambda b: (0, 0)),
            pl.BlockSpec((1, D_MODEL), lambda b: (0, 0)),
            pl.BlockSpec((1, D_MODEL), lambda b: (0, 0)),
            pl.BlockSpec((D_MODEL, D_MODEL), lambda b: (0, 0)),
            pl.BlockSpec((1, D_MODEL), lambda b: (0, 0)),
        ],
        out_specs=pl.BlockSpec((1, N_TOK, D_MODEL), lambda b: (b, 0, 0)),
        out_shape=jax.ShapeDtypeStruct((BS, N_TOK, D_MODEL), jnp.float32),
    )(xc, y, e, wrow, brow, w0, w1, c)
    var_emb, var_idx = _sc_var_outputs(var_emb_table)
    return val_time, var_emb, var_idx
